# baseline (speedup 1.0000x reference)
import jax
import jax.numpy as jnp
from jax.experimental import pallas as pl

_N = 32768
_D = 768
_E = 64
_TOP_K = 8
_TEMP = 1.0

_BLOCK = 512


def _router_body(x_ref, w_ref, scores_ref, weights_ref, idx_ref):
    x = x_ref[...]
    w = w_ref[...]
    s = jax.lax.dot_general(
        x, w, (((1,), (1,)), ((), ())), preferred_element_type=jnp.float32
    )
    scores_ref[...] = s
    st = jax.lax.dot_general(
        w, x, (((1,), (1,)), ((), ())), preferred_element_type=jnp.float32
    )

    expert = jax.lax.broadcasted_iota(jnp.int32, st.shape, 0)
    vals = []
    idxs = []
    for _ in range(_TOP_K):
        top_idx = jnp.argmax(st, axis=0)
        top_val = jnp.max(st, axis=0)
        vals.append(top_val[None, :])
        idxs.append(top_idx[None, :])
        st = jnp.where(expert == top_idx[None, :], -jnp.inf, st)

    top_vals = jnp.concatenate(vals, axis=0)
    top_idxs = jnp.concatenate(idxs, axis=0)
    e = jnp.exp((top_vals - top_vals[0:1, :]) / _TEMP)
    wts = e / jnp.sum(e, axis=0, keepdims=True)
    eye8 = jnp.eye(_TOP_K, dtype=jnp.float32)
    weights_ref[...] = jax.lax.dot_general(
        wts, eye8, (((0,), (0,)), ((), ())), preferred_element_type=jnp.float32
    )
    idx_f = jax.lax.dot_general(
        top_idxs.astype(jnp.float32),
        eye8,
        (((0,), (0,)), ((), ())),
        preferred_element_type=jnp.float32,
    )
    idx_ref[...] = (idx_f + 0.5).astype(jnp.int32)


def kernel(x, W):
    grid = (_N // _BLOCK,)
    scores, weights, indices = pl.pallas_call(
        _router_body,
        grid=grid,
        in_specs=[
            pl.BlockSpec((_BLOCK, _D), lambda i: (i, 0)),
            pl.BlockSpec((_E, _D), lambda i: (0, 0)),
        ],
        out_specs=[
            pl.BlockSpec((_BLOCK, _E), lambda i: (i, 0)),
            pl.BlockSpec((_BLOCK, _TOP_K), lambda i: (i, 0)),
            pl.BlockSpec((_BLOCK, _TOP_K), lambda i: (i, 0)),
        ],
        out_shape=[
            jax.ShapeDtypeStruct((_N, _E), jnp.float32),
            jax.ShapeDtypeStruct((_N, _TOP_K), jnp.float32),
            jax.ShapeDtypeStruct((_N, _TOP_K), jnp.int32),
        ],
    )(x, W)
    return (weights, indices, scores)


# speedup vs baseline: 1.8007x; 1.8007x over previous
import jax
import jax.numpy as jnp
from jax.experimental import pallas as pl

_N = 32768
_D = 768
_E = 64
_TOP_K = 8
_TEMP = 1.0

_BLOCK = 1024


def _router_body(x_ref, w_ref, scores_ref, weights_ref, idx_ref):
    x = x_ref[...]
    w = w_ref[...]
    s = jax.lax.dot_general(
        x, w, (((1,), (1,)), ((), ())), preferred_element_type=jnp.float32
    )
    scores_ref[...] = s
    st = jax.lax.dot_general(
        w, x, (((1,), (1,)), ((), ())), preferred_element_type=jnp.float32
    )

    expert = jax.lax.broadcasted_iota(jnp.int32, st.shape, 0)
    vals = []
    idxs = []
    for _ in range(_TOP_K):
        top_idx = jnp.argmax(st, axis=0)
        top_val = jnp.max(st, axis=0)
        vals.append(top_val[None, :])
        idxs.append(top_idx[None, :])
        st = jnp.where(expert == top_idx[None, :], -jnp.inf, st)

    top_vals = jnp.concatenate(vals, axis=0)
    top_idxs = jnp.concatenate(idxs, axis=0)
    e = jnp.exp((top_vals - top_vals[0:1, :]) / _TEMP)
    weights_ref[...] = e / jnp.sum(e, axis=0, keepdims=True)
    idx_ref[...] = top_idxs


def kernel(x, W):
    grid = (_N // _BLOCK,)
    scores, weights_t, indices_t = pl.pallas_call(
        _router_body,
        grid=grid,
        in_specs=[
            pl.BlockSpec((_BLOCK, _D), lambda i: (i, 0)),
            pl.BlockSpec((_E, _D), lambda i: (0, 0)),
        ],
        out_specs=[
            pl.BlockSpec((_BLOCK, _E), lambda i: (i, 0)),
            pl.BlockSpec((_TOP_K, _BLOCK), lambda i: (0, i)),
            pl.BlockSpec((_TOP_K, _BLOCK), lambda i: (0, i)),
        ],
        out_shape=[
            jax.ShapeDtypeStruct((_N, _E), jnp.float32),
            jax.ShapeDtypeStruct((_TOP_K, _N), jnp.float32),
            jax.ShapeDtypeStruct((_TOP_K, _N), jnp.int32),
        ],
    )(x, W)
    return (weights_t.T, indices_t.T, scores)
